# Initial kernel scaffold; baseline (speedup 1.0000x reference)
import jax
import jax.numpy as jnp
from jax.experimental import pallas as pl


def kernel(sample, keyword_map, pos_items_map, query_table, item_table):
    raise NotImplementedError("write your pallas kernel here")


# speedup vs baseline: 3.6788x; 3.6788x over previous
import jax
import jax.numpy as jnp
from jax import lax
from jax.experimental import pallas as pl
from jax.experimental.pallas import tpu as pltpu
from jax.experimental.pallas import tpu_sc as plsc

EMB = 64
MI = 20
N = 20480
NC, NS = 2, 16
NW = NC * NS
CH = N // NW
SB = 128
NB = CH // SB
IW = CH * MI
IG = 128
NG = IW // IG


def _splat(x):
    return jnp.full((16,), x, jnp.int32)


def _body(flat_hbm, kmap_hbm, pmapt_hbm, qtab_hbm, itab_hbm,
          q_out, i_out, m_out,
          sess_v, qidx_v, icol_v, iflat_v, mask_v, qrows_v, ir0_v, ir1_v,
          zz_v, sem_a, sem0, sem1):
    cid = lax.axis_index("c")
    sid = lax.axis_index("s")
    wid = sid * NC + cid
    iota = lax.iota(jnp.int32, 16)

    pltpu.sync_copy(flat_hbm.at[pl.ds(wid * CH, CH)], sess_v)

    def _orz(t, acc):
        s16 = sess_v[pl.ds(t * 16, 16)]
        return jnp.minimum(acc, s16)
    accz = lax.fori_loop(0, CH // 16, _orz,
                         jnp.full((16,), jnp.int32(2**31 - 1), jnp.int32))
    anyz = jnp.min(accz) == 0

    for b in range(NB):
        idx = sess_v.at[pl.ds(b * SB, SB)]
        cps = [pltpu.async_copy(kmap_hbm.at[idx], qidx_v.at[b], sem_a)]
        for j in range(MI):
            cps.append(pltpu.async_copy(pmapt_hbm.at[j].at[idx],
                                        icol_v.at[b * MI + j], sem_a))
        for c in cps:
            c.wait()

    def _mk(t, carry):
        k = t * 16 + iota
        s = k // MI
        j = k % MI
        row = (s // SB) * MI + j
        col = s % SB
        svec = plsc.load_gather(sess_v, [s])
        it16 = plsc.load_gather(icol_v, [row, col])
        m16 = ((it16 != 0) & (svec != 0)).astype(jnp.int32)
        iflat_v[t >> 3, pl.ds((t & 7) * 16, 16)] = it16
        mask_v[t >> 3, pl.ds((t & 7) * 16, 16)] = m16
        return carry
    lax.fori_loop(0, IW // 16, _mk, 0)
    pltpu.sync_copy(mask_v, m_out.at[wid])

    for b in range(NB):
        pltpu.async_copy(qtab_hbm.at[qidx_v.at[b]], qrows_v, sem_a).wait()
        pltpu.sync_copy(qrows_v, q_out.at[pl.ds(wid * CH + b * SB, SB)])

    bufs = (ir0_v, ir1_v)
    sems = (sem0, sem1)
    prev = None
    for g in range(NG):
        k = g % 2
        cp = pltpu.async_copy(itab_hbm.at[iflat_v.at[g]], bufs[k], sems[k])
        if prev is not None:
            pg, pcp = prev
            pcp.wait()
            pltpu.sync_copy(bufs[pg % 2],
                            i_out.at[pl.ds(wid * IW + pg * IG, IG)])
        prev = (g, cp)
    pg, pcp = prev
    pcp.wait()
    pltpu.sync_copy(bufs[pg % 2], i_out.at[pl.ds(wid * IW + pg * IG, IG)])

    @pl.when(anyz)
    def _zfix():
        for j in range(MI):
            for c4 in range(EMB // 16):
                zz_v[j, pl.ds(c4 * 16, 16)] = jnp.zeros((16,), jnp.float32)

        def _zs(s, carry):
            svec = plsc.load_gather(sess_v, [_splat(s)])
            @pl.when(jnp.max(svec) == 0)
            def _zdo():
                pltpu.sync_copy(zz_v.at[pl.ds(0, 1)],
                                q_out.at[pl.ds(wid * CH + s, 1)])
                pltpu.sync_copy(zz_v,
                                i_out.at[pl.ds((wid * CH + s) * MI, MI)])
            return carry
        lax.fori_loop(0, CH, _zs, 0)


def kernel(sample, keyword_map, pos_items_map, query_table, item_table):
    B, L = sample.shape
    flat1d = sample.reshape(N).astype(jnp.int32)
    kmap = keyword_map.astype(jnp.int32)
    pmapt = jnp.transpose(pos_items_map.astype(jnp.int32))

    mesh = plsc.VectorSubcoreMesh(core_axis_name="c", subcore_axis_name="s")
    out_types = (
        jax.ShapeDtypeStruct((N, EMB), jnp.float32),
        jax.ShapeDtypeStruct((N * MI, EMB), jnp.float32),
        jax.ShapeDtypeStruct((NW, NG, IG), jnp.int32),
    )
    scratch = [
        pltpu.VMEM((CH,), jnp.int32),
        pltpu.VMEM((NB, SB), jnp.int32),
        pltpu.VMEM((NB * MI, SB), jnp.int32),
        pltpu.VMEM((NG, IG), jnp.int32),
        pltpu.VMEM((NG, IG), jnp.int32),
        pltpu.VMEM((SB, EMB), jnp.float32),
        pltpu.VMEM((IG, EMB), jnp.float32),
        pltpu.VMEM((IG, EMB), jnp.float32),
        pltpu.VMEM((MI, EMB), jnp.float32),
        pltpu.SemaphoreType.DMA,
        pltpu.SemaphoreType.DMA,
        pltpu.SemaphoreType.DMA,
    ]
    q, it, m = pl.kernel(_body, out_type=out_types, mesh=mesh,
                         compiler_params=pltpu.CompilerParams(
                             use_tc_tiling_on_sc=False,
                             needs_layout_passes=False),
                         scratch_types=scratch)(
        flat1d, kmap, pmapt, query_table, item_table)
    return (q.reshape(B, L, EMB),
            it.reshape(B, L, MI, EMB),
            (m.reshape(B, L, MI) != 0))


# speedup vs baseline: 3.7806x; 1.0277x over previous
import jax
import jax.numpy as jnp
from jax import lax
from jax.experimental import pallas as pl
from jax.experimental.pallas import tpu as pltpu
from jax.experimental.pallas import tpu_sc as plsc

EMB = 64
MI = 20
N = 20480
NC, NS = 2, 16
NW = NC * NS
CH = N // NW
SB = 128
NB = CH // SB
IW = CH * MI
IG = 128
NG = IW // IG


def _splat(x):
    return jnp.full((16,), x, jnp.int32)


def _body(flat_hbm, kmap_hbm, pmapt_hbm, qtab_hbm, itab_hbm,
          q_out, i_out, m_out,
          sess_v, qidx_v, icol_v, iflat_v, mask_v, qrows_v,
          ir0_v, ir1_v, ir2_v, ir3_v, zz_v,
          sem_a, sem0, sem1, sem2, sem3, osem0, osem1, osem2, osem3):
    cid = lax.axis_index("c")
    sid = lax.axis_index("s")
    wid = sid * NC + cid
    iota = lax.iota(jnp.int32, 16)

    pltpu.sync_copy(flat_hbm.at[pl.ds(wid * CH, CH)], sess_v)

    def _orz(t, acc):
        s16 = sess_v[pl.ds(t * 16, 16)]
        return jnp.minimum(acc, s16)
    accz = lax.fori_loop(0, CH // 16, _orz,
                         jnp.full((16,), jnp.int32(2**31 - 1), jnp.int32))
    anyz = jnp.min(accz) == 0

    for b in range(NB):
        idx = sess_v.at[pl.ds(b * SB, SB)]
        cps = [pltpu.async_copy(kmap_hbm.at[idx], qidx_v.at[b], sem_a)]
        for j in range(MI):
            cps.append(pltpu.async_copy(pmapt_hbm.at[j].at[idx],
                                        icol_v.at[b * MI + j], sem_a))
        for c in cps:
            c.wait()

    def _mk(t, carry):
        k = t * 16 + iota
        s = k // MI
        j = k % MI
        row = (s // SB) * MI + j
        col = s % SB
        svec = plsc.load_gather(sess_v, [s])
        it16 = plsc.load_gather(icol_v, [row, col])
        m16 = ((it16 != 0) & (svec != 0)).astype(jnp.int32)
        iflat_v[t >> 3, pl.ds((t & 7) * 16, 16)] = it16
        mask_v[t >> 3, pl.ds((t & 7) * 16, 16)] = m16
        return carry
    lax.fori_loop(0, IW // 16, _mk, 0)
    pltpu.sync_copy(mask_v, m_out.at[wid])

    for b in range(NB):
        pltpu.async_copy(qtab_hbm.at[qidx_v.at[b]], qrows_v, sem_a).wait()
        pltpu.sync_copy(qrows_v, q_out.at[pl.ds(wid * CH + b * SB, SB)])

    bufs = (ir0_v, ir1_v, ir2_v, ir3_v)
    gsems = (sem0, sem1, sem2, sem3)
    osems = (osem0, osem1, osem2, osem3)
    gcp = [None] * 4
    ocp = [None] * 4

    def _ostart(g):
        k = g % 4
        return pltpu.async_copy(bufs[k],
                                i_out.at[pl.ds(wid * IW + g * IG, IG)],
                                osems[k])

    def _gstart(g):
        k = g % 4
        return pltpu.async_copy(itab_hbm.at[iflat_v.at[g]], bufs[k], gsems[k])

    gcp[0] = _gstart(0)
    gcp[1] = _gstart(1)
    for g in range(NG):
        k = g % 4
        if g + 2 < NG:
            kk = (g + 2) % 4
            if ocp[kk] is not None:
                ocp[kk].wait()
                ocp[kk] = None
            gcp[kk] = _gstart(g + 2)
        gcp[k].wait()
        ocp[k] = _ostart(g)
    for k in range(4):
        if ocp[k] is not None:
            ocp[k].wait()

    @pl.when(anyz)
    def _zfix():
        for j in range(MI):
            for c4 in range(EMB // 16):
                zz_v[j, pl.ds(c4 * 16, 16)] = jnp.zeros((16,), jnp.float32)

        def _zs(s, carry):
            svec = plsc.load_gather(sess_v, [_splat(s)])
            @pl.when(jnp.max(svec) == 0)
            def _zdo():
                pltpu.sync_copy(zz_v.at[pl.ds(0, 1)],
                                q_out.at[pl.ds(wid * CH + s, 1)])
                pltpu.sync_copy(zz_v,
                                i_out.at[pl.ds((wid * CH + s) * MI, MI)])
            return carry
        lax.fori_loop(0, CH, _zs, 0)


def kernel(sample, keyword_map, pos_items_map, query_table, item_table):
    B, L = sample.shape
    flat1d = sample.reshape(N).astype(jnp.int32)
    kmap = keyword_map.astype(jnp.int32)
    pmapt = jnp.transpose(pos_items_map.astype(jnp.int32))

    mesh = plsc.VectorSubcoreMesh(core_axis_name="c", subcore_axis_name="s")
    out_types = (
        jax.ShapeDtypeStruct((N, EMB), jnp.float32),
        jax.ShapeDtypeStruct((N * MI, EMB), jnp.float32),
        jax.ShapeDtypeStruct((NW, NG, IG), jnp.int32),
    )
    scratch = [
        pltpu.VMEM((CH,), jnp.int32),
        pltpu.VMEM((NB, SB), jnp.int32),
        pltpu.VMEM((NB * MI, SB), jnp.int32),
        pltpu.VMEM((NG, IG), jnp.int32),
        pltpu.VMEM((NG, IG), jnp.int32),
        pltpu.VMEM((SB, EMB), jnp.float32),
        pltpu.VMEM((IG, EMB), jnp.float32),
        pltpu.VMEM((IG, EMB), jnp.float32),
        pltpu.VMEM((IG, EMB), jnp.float32),
        pltpu.VMEM((IG, EMB), jnp.float32),
        pltpu.VMEM((MI, EMB), jnp.float32),
    ] + [pltpu.SemaphoreType.DMA] * 9
    q, it, m = pl.kernel(_body, out_type=out_types, mesh=mesh,
                         compiler_params=pltpu.CompilerParams(
                             use_tc_tiling_on_sc=False,
                             needs_layout_passes=False),
                         scratch_types=scratch)(
        flat1d, kmap, pmapt, query_table, item_table)
    return (q.reshape(B, L, EMB),
            it.reshape(B, L, MI, EMB),
            (m.reshape(B, L, MI) != 0))


# speedup vs baseline: 3.8226x; 1.0111x over previous
import jax
import jax.numpy as jnp
from jax import lax
from jax.experimental import pallas as pl
from jax.experimental.pallas import tpu as pltpu
from jax.experimental.pallas import tpu_sc as plsc

EMB = 64
MI = 20
N = 20480
NC, NS = 2, 16
NW = NC * NS
CH = N // NW
SB = 128
NB = CH // SB
IW = CH * MI
IG = 128
NG = IW // IG


def _splat(x):
    return jnp.full((16,), x, jnp.int32)


def _body(flat_hbm, kmap_hbm, pmapt_hbm, qtab_hbm, itab_hbm,
          q_out, i_out, m_out,
          sess_v, qidx_v, icol_v, iflat_v, mask_v, qrows5_v,
          ir0_v, ir1_v, ir2_v, ir3_v, zz_v,
          sem_a, sem_q, sem0, sem1, sem2, sem3,
          osem0, osem1, osem2, osem3):
    cid = lax.axis_index("c")
    sid = lax.axis_index("s")
    wid = sid * NC + cid
    iota = lax.iota(jnp.int32, 16)

    pltpu.sync_copy(flat_hbm.at[pl.ds(wid * CH, CH)], sess_v)

    def _orz(t, acc):
        s16 = sess_v[pl.ds(t * 16, 16)]
        return jnp.minimum(acc, s16)
    accz = lax.fori_loop(0, CH // 16, _orz,
                         jnp.full((16,), jnp.int32(2**31 - 1), jnp.int32))
    anyz = jnp.min(accz) == 0

    h1prev = None
    for b in range(NB):
        idx = sess_v.at[pl.ds(b * SB, SB)]
        cps = [pltpu.async_copy(kmap_hbm.at[idx], qidx_v.at[b], sem_a)]
        for j in range(MI):
            cps.append(pltpu.async_copy(pmapt_hbm.at[j].at[idx],
                                        icol_v.at[b * MI + j], sem_a))
        if h1prev is not None:
            for c in h1prev:
                c.wait()
        h1prev = cps
    for c in h1prev:
        c.wait()

    def _mk(t, carry):
        k = t * 16 + iota
        s = k // MI
        j = k % MI
        row = (s // SB) * MI + j
        col = s % SB
        svec = plsc.load_gather(sess_v, [s])
        it16 = plsc.load_gather(icol_v, [row, col])
        m16 = ((it16 != 0) & (svec != 0)).astype(jnp.int32)
        iflat_v[t >> 3, pl.ds((t & 7) * 16, 16)] = it16
        mask_v[t >> 3, pl.ds((t & 7) * 16, 16)] = m16
        return carry
    qcps = [pltpu.async_copy(qtab_hbm.at[qidx_v.at[b]], qrows5_v.at[b],
                             sem_q)
            for b in range(NB)]
    lax.fori_loop(0, IW // 16, _mk, 0)
    pltpu.sync_copy(mask_v, m_out.at[wid])

    for b in range(NB):
        qcps[b].wait()
        pltpu.sync_copy(qrows5_v.at[b], q_out.at[pl.ds(wid * CH + b * SB, SB)])

    bufs = (ir0_v, ir1_v, ir2_v, ir3_v)
    gsems = (sem0, sem1, sem2, sem3)
    osems = (osem0, osem1, osem2, osem3)
    gcp = [None] * 4
    ocp = [None] * 4

    def _ostart(g):
        k = g % 4
        return pltpu.async_copy(bufs[k],
                                i_out.at[pl.ds(wid * IW + g * IG, IG)],
                                osems[k])

    def _gstart(g):
        k = g % 4
        return pltpu.async_copy(itab_hbm.at[iflat_v.at[g]], bufs[k], gsems[k])

    gcp[0] = _gstart(0)
    gcp[1] = _gstart(1)
    for g in range(NG):
        k = g % 4
        if g + 2 < NG:
            kk = (g + 2) % 4
            if ocp[kk] is not None:
                ocp[kk].wait()
                ocp[kk] = None
            gcp[kk] = _gstart(g + 2)
        gcp[k].wait()
        ocp[k] = _ostart(g)
    for k in range(4):
        if ocp[k] is not None:
            ocp[k].wait()

    @pl.when(anyz)
    def _zfix():
        for j in range(MI):
            for c4 in range(EMB // 16):
                zz_v[j, pl.ds(c4 * 16, 16)] = jnp.zeros((16,), jnp.float32)

        def _zs(s, carry):
            svec = plsc.load_gather(sess_v, [_splat(s)])
            @pl.when(jnp.max(svec) == 0)
            def _zdo():
                pltpu.sync_copy(zz_v.at[pl.ds(0, 1)],
                                q_out.at[pl.ds(wid * CH + s, 1)])
                pltpu.sync_copy(zz_v,
                                i_out.at[pl.ds((wid * CH + s) * MI, MI)])
            return carry
        lax.fori_loop(0, CH, _zs, 0)


def kernel(sample, keyword_map, pos_items_map, query_table, item_table):
    B, L = sample.shape
    flat1d = sample.reshape(N).astype(jnp.int32)
    kmap = keyword_map.astype(jnp.int32)
    pmapt = jnp.transpose(pos_items_map.astype(jnp.int32))

    mesh = plsc.VectorSubcoreMesh(core_axis_name="c", subcore_axis_name="s")
    out_types = (
        jax.ShapeDtypeStruct((N, EMB), jnp.float32),
        jax.ShapeDtypeStruct((N * MI, EMB), jnp.float32),
        jax.ShapeDtypeStruct((NW, NG, IG), jnp.int32),
    )
    scratch = [
        pltpu.VMEM((CH,), jnp.int32),
        pltpu.VMEM((NB, SB), jnp.int32),
        pltpu.VMEM((NB * MI, SB), jnp.int32),
        pltpu.VMEM((NG, IG), jnp.int32),
        pltpu.VMEM((NG, IG), jnp.int32),
        pltpu.VMEM((NB, SB, EMB), jnp.float32),
        pltpu.VMEM((IG, EMB), jnp.float32),
        pltpu.VMEM((IG, EMB), jnp.float32),
        pltpu.VMEM((IG, EMB), jnp.float32),
        pltpu.VMEM((IG, EMB), jnp.float32),
        pltpu.VMEM((MI, EMB), jnp.float32),
    ] + [pltpu.SemaphoreType.DMA] * 10
    q, it, m = pl.kernel(_body, out_type=out_types, mesh=mesh,
                         compiler_params=pltpu.CompilerParams(
                             use_tc_tiling_on_sc=False,
                             needs_layout_passes=False),
                         scratch_types=scratch)(
        flat1d, kmap, pmapt, query_table, item_table)
    return (q.reshape(B, L, EMB),
            it.reshape(B, L, MI, EMB),
            (m.reshape(B, L, MI) != 0))
